# baseline (speedup 1.0000x reference)
import jax
import jax.numpy as jnp
from jax import lax
from jax.experimental import pallas as pl
from jax.experimental.pallas import tpu as pltpu

_T, _B, _NPFO, _NH = 32, 64, 1280, 500000
_EPS = 1e-08
_C = 8192
_NBLK = (_NH + _C - 1) // _C
_RC = 256
_JC = 256


def _softplus_bce(x, y):
    return jnp.maximum(x, 0.0) - x * y + jnp.log1p(jnp.exp(-jnp.abs(x)))


def _body(lg_ref, hbc_ref, htp_ref, gtbr_ref, gtbc_ref, gmom_ref, gpm_ref,
          gpid_ref, gch_ref, pmom_ref, ppm_ref, ppid_ref, pch_ref, stop_ref,
          out_ref, acc_ref, pperow_ref):
    i = pl.program_id(0)

    @pl.when(i == 0)
    def _small_terms():
        gtbc = gtbc_ref[...]
        b_iota_r = lax.broadcasted_iota(jnp.int32, (_NPFO, _B), 1)
        ppe_row = jnp.sum((gtbc == b_iota_r).astype(jnp.float32), axis=0,
                          keepdims=True)
        pperow_ref[...] = jnp.broadcast_to(ppe_row, (8, _B))
        acc_ref[0] = 0.0
        acc_ref[1] = 0.0
        gtbr = gtbr_ref[...]
        step_row = jnp.zeros((1, _NPFO), jnp.float32)
        for jc in range(0, _NPFO, _JC):
            gc = gtbc[jc:jc + _JC, :]
            eq = (gc == gtbr)
            jlt = ((jc + lax.broadcasted_iota(jnp.int32, (_JC, _NPFO), 0))
                   < lax.broadcasted_iota(jnp.int32, (_JC, _NPFO), 1))
            step_row = step_row + jnp.sum((eq & jlt).astype(jnp.float32),
                                          axis=0, keepdims=True)
        valid_r = step_row < float(_T)
        c_row = (jnp.minimum(step_row, float(_T - 1)) * float(_B)
                 + gtbr.astype(jnp.float32))

        mom = gmom_ref[...]
        nrm = jnp.sqrt(jnp.sum(mom * mom, axis=1, keepdims=True))
        gdir = mom / jnp.maximum(nrm, _EPS)
        x5 = gpid_ref[...]
        m5 = jnp.max(x5, axis=1, keepdims=True)
        li = lax.broadcasted_iota(jnp.int32, (_NPFO, 5), 1)
        idx5 = jnp.min(jnp.where(x5 == m5, li, 5), axis=1, keepdims=True)
        oh5 = (li == idx5).astype(jnp.float32)
        packed = jnp.concatenate(
            [gdir, gpm_ref[...], oh5, gch_ref[...]], axis=1)

        ppe_col = jnp.sum(
            (lax.broadcasted_iota(jnp.int32, (_B, _NPFO), 0) == gtbr)
            .astype(jnp.float32), axis=1, keepdims=True)

        nd = nm = npd = nc = dv = ss = 0.0
        for rb in range(0, _T * _B, _RC):
            rid0 = (rb + lax.broadcasted_iota(jnp.int32, (_RC, _NPFO), 0))
            mm = ((rid0.astype(jnp.float32) == c_row) & valid_r)
            g = lax.dot_general(mm.astype(jnp.float32), packed,
                                (((1,), (0,)), ((), ())),
                                preferred_element_type=jnp.float32)
            rid1 = rb + lax.broadcasted_iota(jnp.int32, (_RC, 1), 0)
            t_c = lax.shift_right_logical(rid1, 6).astype(jnp.float32)
            ohb2 = (lax.broadcasted_iota(jnp.int32, (_RC, _B), 1)
                    == jnp.bitwise_and(rid1, _B - 1)).astype(jnp.float32)
            ppe_c = lax.dot_general(ohb2, ppe_col, (((1,), (0,)), ((), ())),
                                    preferred_element_type=jnp.float32)
            vf = (t_c < ppe_c).astype(jnp.float32)
            dv += jnp.sum(vf)
            pm = pmom_ref[rb:rb + _RC, :]
            pn = jnp.sqrt(jnp.sum(pm * pm, axis=1, keepdims=True))
            pdir = pm / jnp.maximum(pn, _EPS)
            cos = jnp.sum(pdir * g[:, 0:3], axis=1, keepdims=True)
            nd += jnp.sum((1.0 - cos) * vf)
            dpm = ppm_ref[rb:rb + _RC, :] - g[:, 3:4]
            nm += jnp.sum(dpm * dpm * vf)
            xp = ppid_ref[rb:rb + _RC, :]
            mx = jnp.max(xp, axis=1, keepdims=True)
            lse = mx + jnp.log(jnp.sum(jnp.exp(xp - mx), axis=1,
                                       keepdims=True))
            ce = -jnp.sum((xp - lse) * g[:, 4:9], axis=1, keepdims=True)
            npd += jnp.sum(ce * vf)
            dch = pch_ref[rb:rb + _RC, :] - g[:, 9:10]
            nc += jnp.sum(dch * dch * vf)
            sx = stop_ref[rb:rb + _RC, :]
            ss += jnp.sum(_softplus_bce(sx, 1.0 - vf))
        acc_ref[2] = nd
        acc_ref[3] = nm
        acc_ref[4] = npd
        acc_ref[5] = nc
        acc_ref[6] = dv
        acc_ref[7] = ss

    x_raw = lg_ref[...]
    htp = htp_ref[...]
    hbc = hbc_ref[...]
    inb = (i * _C + lax.broadcasted_iota(jnp.int32, (1, _C), 1)) < _NH
    x = jnp.where(inb, x_raw, 0.0)
    sp = jnp.maximum(x, 0.0) + jnp.log1p(jnp.exp(-jnp.abs(x)))
    spb = sp.astype(jnp.bfloat16)
    t_row = lax.broadcasted_iota(jnp.int32, (_T, _C), 0)
    xmb = x.astype(jnp.bfloat16) * (htp == t_row).astype(jnp.bfloat16)
    inbc = (i * _C + lax.broadcasted_iota(jnp.int32, (_C, 1), 0)) < _NH
    oht = ((lax.broadcasted_iota(jnp.int32, (_C, _B), 1) == hbc)
           & inbc).astype(jnp.bfloat16)
    dn = (((1,), (0,)), ((), ()))
    s_tb = lax.dot_general(spb, oht, dn,
                           preferred_element_type=jnp.float32)
    g_tb = lax.dot_general(xmb, oht, dn,
                           preferred_element_type=jnp.float32)
    cnt8 = lax.dot_general(
        jnp.broadcast_to(inb, (8, _C)).astype(jnp.bfloat16), oht, dn,
        preferred_element_type=jnp.float32)
    ppe_r = pperow_ref[0:1, :]
    vt = (lax.broadcasted_iota(jnp.int32, (_T, _B), 0).astype(jnp.float32)
          < ppe_r).astype(jnp.float32)
    contrib = jnp.sum((s_tb - g_tb) * vt)
    dcontrib = jnp.sum(cnt8[0:1, :] * jnp.minimum(ppe_r, float(_T)))
    acc_ref[0] += contrib
    acc_ref[1] += dcontrib

    @pl.when(i == _NBLK - 1)
    def _finalize():
        den = jnp.maximum(acc_ref[6], 1.0)
        l_dir = acc_ref[2] / den
        l_mag = acc_ref[3] / den
        l_pid = acc_ref[4] / den
        l_chg = acc_ref[5] / den
        l_asn = acc_ref[0] / jnp.maximum(acc_ref[1], 1.0)
        l_stp = acc_ref[7] / float(_T * _B)
        out_ref[0] = (l_dir + l_mag + l_pid + 0.5 * l_chg + l_asn
                      + 0.5 * l_stp)
        out_ref[1] = l_dir
        out_ref[2] = l_mag
        out_ref[3] = l_pid
        out_ref[4] = l_chg
        out_ref[5] = l_asn
        out_ref[6] = l_stp


def kernel(pfo_momentum, pfo_p_mod, pfo_pid, pfo_charge, assignments,
           assignments_logits, stop_logits, gt_momentum, gt_p_mod, gt_pid,
           gt_charge, gt_batch, hit_to_pfo, hit_batch):
    del assignments
    lg = assignments_logits.reshape(_T, _NH)
    hbc = hit_batch.astype(jnp.int32).reshape(_NH, 1)
    htp2 = hit_to_pfo.astype(jnp.int32).reshape(1, _NH)
    gtb = gt_batch.astype(jnp.int32)
    gtbr = gtb.reshape(1, _NPFO)
    gtbc = gtb.reshape(_NPFO, 1)
    pmom = pfo_momentum.reshape(_T * _B, 3)
    ppm = pfo_p_mod.reshape(_T * _B, 1)
    ppid = pfo_pid.reshape(_T * _B, 5)
    pch = pfo_charge.reshape(_T * _B, 1)
    stp = stop_logits.reshape(_T * _B, 1)

    full = lambda s: pl.BlockSpec(s, lambda i: (0, 0))
    out = pl.pallas_call(
        _body,
        grid=(_NBLK,),
        in_specs=[
            pl.BlockSpec((_T, _C), lambda i: (0, i)),
            pl.BlockSpec((_C, 1), lambda i: (i, 0)),
            pl.BlockSpec((1, _C), lambda i: (0, i)),
            full((1, _NPFO)),
            full((_NPFO, 1)),
            full((_NPFO, 3)),
            full((_NPFO, 1)),
            full((_NPFO, 5)),
            full((_NPFO, 1)),
            full((_T * _B, 3)),
            full((_T * _B, 1)),
            full((_T * _B, 5)),
            full((_T * _B, 1)),
            full((_T * _B, 1)),
        ],
        out_specs=pl.BlockSpec(memory_space=pltpu.SMEM),
        out_shape=jax.ShapeDtypeStruct((8,), jnp.float32),
        scratch_shapes=[pltpu.SMEM((8,), jnp.float32),
                        pltpu.VMEM((8, _B), jnp.float32)],
        compiler_params=pltpu.CompilerParams(
            dimension_semantics=("arbitrary",)),
    )(lg, hbc, htp2, gtbr, gtbc, gt_momentum, gt_p_mod, gt_pid, gt_charge,
      pmom, ppm, ppid, pch, stp)
    return (out[0], out[1], out[2], out[3], out[4], out[5], out[6])


# speedup vs baseline: 2.1087x; 2.1087x over previous
import jax
import jax.numpy as jnp
from jax import lax
from jax.experimental import pallas as pl
from jax.experimental.pallas import tpu as pltpu

_T, _B, _NPFO, _NH = 32, 64, 1280, 500000
_EPS = 1e-08
_C = 8192
_NBLK = (_NH + _C - 1) // _C
_RC = 256
_JC = 256


def _softplus_bce(x, y):
    return jnp.maximum(x, 0.0) - x * y + jnp.log1p(jnp.exp(-jnp.abs(x)))


def _body(lg_ref, hbc_ref, htp_ref, gtbr_ref, gtbc_ref, gmom_ref, gpm_ref,
          gpid_ref, gch_ref, pmom_ref, ppm_ref, ppid_ref, pch_ref, stop_ref,
          out_ref, acc_ref, pperow_ref):
    i = pl.program_id(0)

    @pl.when(i == 0)
    def _small_terms():
        gtbc = gtbc_ref[...]
        b_iota_r = lax.broadcasted_iota(jnp.int32, (_NPFO, _B), 1)
        ppe_row = jnp.sum((gtbc == b_iota_r).astype(jnp.float32), axis=0,
                          keepdims=True)
        pperow_ref[...] = jnp.broadcast_to(ppe_row, (8, _B))
        acc_ref[0] = 0.0
        acc_ref[1] = 0.0
        gtbr = gtbr_ref[...]
        step_row = jnp.zeros((1, _NPFO), jnp.float32)
        for jc in range(0, _NPFO, _JC):
            gc = gtbc[jc:jc + _JC, :]
            eq = (gc == gtbr)
            jlt = ((jc + lax.broadcasted_iota(jnp.int32, (_JC, _NPFO), 0))
                   < lax.broadcasted_iota(jnp.int32, (_JC, _NPFO), 1))
            step_row = step_row + jnp.sum((eq & jlt).astype(jnp.float32),
                                          axis=0, keepdims=True)
        valid_r = step_row < float(_T)
        c_row = (jnp.minimum(step_row, float(_T - 1)) * float(_B)
                 + gtbr.astype(jnp.float32))

        mom = gmom_ref[...]
        nrm = jnp.sqrt(jnp.sum(mom * mom, axis=1, keepdims=True))
        gdir = mom / jnp.maximum(nrm, _EPS)
        x5 = gpid_ref[...]
        m5 = jnp.max(x5, axis=1, keepdims=True)
        li = lax.broadcasted_iota(jnp.int32, (_NPFO, 5), 1)
        idx5 = jnp.min(jnp.where(x5 == m5, li, 5), axis=1, keepdims=True)
        oh5 = (li == idx5).astype(jnp.float32)
        packed = jnp.concatenate(
            [gdir, gpm_ref[...], oh5, gch_ref[...]], axis=1)

        ppe_col = jnp.sum(
            (lax.broadcasted_iota(jnp.int32, (_B, _NPFO), 0) == gtbr)
            .astype(jnp.float32), axis=1, keepdims=True)

        nd = nm = npd = nc = dv = ss = 0.0
        for rb in range(0, _T * _B, _RC):
            rid0 = (rb + lax.broadcasted_iota(jnp.int32, (_RC, _NPFO), 0))
            mm = ((rid0.astype(jnp.float32) == c_row) & valid_r)
            g = lax.dot_general(mm.astype(jnp.float32), packed,
                                (((1,), (0,)), ((), ())),
                                preferred_element_type=jnp.float32)
            rid1 = rb + lax.broadcasted_iota(jnp.int32, (_RC, 1), 0)
            t_c = lax.shift_right_logical(rid1, 6).astype(jnp.float32)
            ohb2 = (lax.broadcasted_iota(jnp.int32, (_RC, _B), 1)
                    == jnp.bitwise_and(rid1, _B - 1)).astype(jnp.float32)
            ppe_c = lax.dot_general(ohb2, ppe_col, (((1,), (0,)), ((), ())),
                                    preferred_element_type=jnp.float32)
            vf = (t_c < ppe_c).astype(jnp.float32)
            dv += jnp.sum(vf)
            pm = pmom_ref[rb:rb + _RC, :]
            pn = jnp.sqrt(jnp.sum(pm * pm, axis=1, keepdims=True))
            pdir = pm / jnp.maximum(pn, _EPS)
            cos = jnp.sum(pdir * g[:, 0:3], axis=1, keepdims=True)
            nd += jnp.sum((1.0 - cos) * vf)
            dpm = ppm_ref[rb:rb + _RC, :] - g[:, 3:4]
            nm += jnp.sum(dpm * dpm * vf)
            xp = ppid_ref[rb:rb + _RC, :]
            mx = jnp.max(xp, axis=1, keepdims=True)
            lse = mx + jnp.log(jnp.sum(jnp.exp(xp - mx), axis=1,
                                       keepdims=True))
            ce = -jnp.sum((xp - lse) * g[:, 4:9], axis=1, keepdims=True)
            npd += jnp.sum(ce * vf)
            dch = pch_ref[rb:rb + _RC, :] - g[:, 9:10]
            nc += jnp.sum(dch * dch * vf)
            sx = stop_ref[rb:rb + _RC, :]
            ss += jnp.sum(_softplus_bce(sx, 1.0 - vf))
        acc_ref[2] = nd
        acc_ref[3] = nm
        acc_ref[4] = npd
        acc_ref[5] = nc
        acc_ref[6] = dv
        acc_ref[7] = ss

    x = lg_ref[...]
    hb = hbc_ref[...]
    htp = htp_ref[...]
    inb = (i * _C + lax.broadcasted_iota(jnp.int32, (1, _C), 1)) < _NH
    ohb = (lax.broadcasted_iota(jnp.int32, (_B, _C), 0) == hb)
    ppe_h = lax.dot_general(pperow_ref[0:1, :], ohb.astype(jnp.float32),
                            (((1,), (0,)), ((), ())),
                            preferred_element_type=jnp.float32)
    t_row = lax.broadcasted_iota(jnp.int32, (_T, _C), 0)
    valid = (t_row.astype(jnp.float32) < ppe_h) & inb
    y = (htp == t_row) & valid
    sp = jnp.maximum(x, 0.0) + jnp.log1p(jnp.exp(-jnp.abs(x)))
    contrib = (jnp.sum(jnp.where(valid, sp, 0.0))
               - jnp.sum(jnp.where(y, x, 0.0)))
    dcontrib = jnp.sum(valid.astype(jnp.float32))
    acc_ref[0] += contrib
    acc_ref[1] += dcontrib

    @pl.when(i == _NBLK - 1)
    def _finalize():
        den = jnp.maximum(acc_ref[6], 1.0)
        l_dir = acc_ref[2] / den
        l_mag = acc_ref[3] / den
        l_pid = acc_ref[4] / den
        l_chg = acc_ref[5] / den
        l_asn = acc_ref[0] / jnp.maximum(acc_ref[1], 1.0)
        l_stp = acc_ref[7] / float(_T * _B)
        out_ref[0] = (l_dir + l_mag + l_pid + 0.5 * l_chg + l_asn
                      + 0.5 * l_stp)
        out_ref[1] = l_dir
        out_ref[2] = l_mag
        out_ref[3] = l_pid
        out_ref[4] = l_chg
        out_ref[5] = l_asn
        out_ref[6] = l_stp


def kernel(pfo_momentum, pfo_p_mod, pfo_pid, pfo_charge, assignments,
           assignments_logits, stop_logits, gt_momentum, gt_p_mod, gt_pid,
           gt_charge, gt_batch, hit_to_pfo, hit_batch):
    del assignments
    lg = assignments_logits.reshape(_T, _NH)
    hbc = hit_batch.astype(jnp.int32).reshape(1, _NH)
    htp2 = hit_to_pfo.astype(jnp.int32).reshape(1, _NH)
    gtb = gt_batch.astype(jnp.int32)
    gtbr = gtb.reshape(1, _NPFO)
    gtbc = gtb.reshape(_NPFO, 1)
    pmom = pfo_momentum.reshape(_T * _B, 3)
    ppm = pfo_p_mod.reshape(_T * _B, 1)
    ppid = pfo_pid.reshape(_T * _B, 5)
    pch = pfo_charge.reshape(_T * _B, 1)
    stp = stop_logits.reshape(_T * _B, 1)

    full = lambda s: pl.BlockSpec(s, lambda i: (0, 0))
    out = pl.pallas_call(
        _body,
        grid=(_NBLK,),
        in_specs=[
            pl.BlockSpec((_T, _C), lambda i: (0, i)),
            pl.BlockSpec((1, _C), lambda i: (0, i)),
            pl.BlockSpec((1, _C), lambda i: (0, i)),
            full((1, _NPFO)),
            full((_NPFO, 1)),
            full((_NPFO, 3)),
            full((_NPFO, 1)),
            full((_NPFO, 5)),
            full((_NPFO, 1)),
            full((_T * _B, 3)),
            full((_T * _B, 1)),
            full((_T * _B, 5)),
            full((_T * _B, 1)),
            full((_T * _B, 1)),
        ],
        out_specs=pl.BlockSpec(memory_space=pltpu.SMEM),
        out_shape=jax.ShapeDtypeStruct((8,), jnp.float32),
        scratch_shapes=[pltpu.SMEM((8,), jnp.float32),
                        pltpu.VMEM((8, _B), jnp.float32)],
        compiler_params=pltpu.CompilerParams(
            dimension_semantics=("arbitrary",)),
    )(lg, hbc, htp2, gtbr, gtbc, gt_momentum, gt_p_mod, gt_pid, gt_charge,
      pmom, ppm, ppid, pch, stp)
    return (out[0], out[1], out[2], out[3], out[4], out[5], out[6])


# speedup vs baseline: 2.4539x; 1.1637x over previous
import jax
import jax.numpy as jnp
from jax import lax
from jax.experimental import pallas as pl
from jax.experimental.pallas import tpu as pltpu

_T, _B, _NPFO, _NH = 32, 64, 1280, 500000
_EPS = 1e-08
_C = 8192
_NBLK = (_NH + _C - 1) // _C
_RC = 256
_JC = 256


def _softplus_bce(x, y):
    return jnp.maximum(x, 0.0) - x * y + jnp.log1p(jnp.exp(-jnp.abs(x)))


def _body(lg_ref, hbc_ref, htp_ref, gtbr_ref, gtbc_ref, gmom_ref, gpm_ref,
          gpid_ref, gch_ref, pmom_ref, ppm_ref, ppid_ref, pch_ref, stop_ref,
          out_ref, acc_ref, pperow_ref):
    i = pl.program_id(0)

    @pl.when(i == 0)
    def _small_terms():
        gtbc = gtbc_ref[...]
        b_iota_r = lax.broadcasted_iota(jnp.int32, (_NPFO, _B), 1)
        ppe_row = jnp.sum((gtbc == b_iota_r).astype(jnp.float32), axis=0,
                          keepdims=True)
        pperow_ref[...] = jnp.broadcast_to(ppe_row, (8, _B))
        acc_ref[0] = 0.0
        acc_ref[1] = 0.0
        gtbr = gtbr_ref[...]
        step_row = jnp.zeros((1, _NPFO), jnp.float32)
        for jc in range(0, _NPFO, _JC):
            gc = gtbc[jc:jc + _JC, :]
            eq = (gc == gtbr)
            jlt = ((jc + lax.broadcasted_iota(jnp.int32, (_JC, _NPFO), 0))
                   < lax.broadcasted_iota(jnp.int32, (_JC, _NPFO), 1))
            step_row = step_row + jnp.sum((eq & jlt).astype(jnp.float32),
                                          axis=0, keepdims=True)
        valid_r = step_row < float(_T)
        c_row = (jnp.minimum(step_row, float(_T - 1)) * float(_B)
                 + gtbr.astype(jnp.float32))

        mom = gmom_ref[...]
        nrm = jnp.sqrt(jnp.sum(mom * mom, axis=1, keepdims=True))
        gdir = mom / jnp.maximum(nrm, _EPS)
        x5 = gpid_ref[...]
        m5 = jnp.max(x5, axis=1, keepdims=True)
        li = lax.broadcasted_iota(jnp.int32, (_NPFO, 5), 1)
        idx5 = jnp.min(jnp.where(x5 == m5, li, 5), axis=1, keepdims=True)
        oh5 = (li == idx5).astype(jnp.float32)
        packed = jnp.concatenate(
            [gdir, gpm_ref[...], oh5, gch_ref[...]], axis=1)

        ppe_col = jnp.sum(
            (lax.broadcasted_iota(jnp.int32, (_B, _NPFO), 0) == gtbr)
            .astype(jnp.float32), axis=1, keepdims=True)

        nd = nm = npd = nc = dv = ss = 0.0
        for rb in range(0, _T * _B, _RC):
            rid0 = (rb + lax.broadcasted_iota(jnp.int32, (_RC, _NPFO), 0))
            mm = ((rid0.astype(jnp.float32) == c_row) & valid_r)
            g = lax.dot_general(mm.astype(jnp.float32), packed,
                                (((1,), (0,)), ((), ())),
                                preferred_element_type=jnp.float32)
            rid1 = rb + lax.broadcasted_iota(jnp.int32, (_RC, 1), 0)
            t_c = lax.shift_right_logical(rid1, 6).astype(jnp.float32)
            ohb2 = (lax.broadcasted_iota(jnp.int32, (_RC, _B), 1)
                    == jnp.bitwise_and(rid1, _B - 1)).astype(jnp.float32)
            ppe_c = lax.dot_general(ohb2, ppe_col, (((1,), (0,)), ((), ())),
                                    preferred_element_type=jnp.float32)
            vf = (t_c < ppe_c).astype(jnp.float32)
            dv += jnp.sum(vf)
            pm = pmom_ref[rb:rb + _RC, :]
            pn = jnp.sqrt(jnp.sum(pm * pm, axis=1, keepdims=True))
            pdir = pm / jnp.maximum(pn, _EPS)
            cos = jnp.sum(pdir * g[:, 0:3], axis=1, keepdims=True)
            nd += jnp.sum((1.0 - cos) * vf)
            dpm = ppm_ref[rb:rb + _RC, :] - g[:, 3:4]
            nm += jnp.sum(dpm * dpm * vf)
            xp = ppid_ref[rb:rb + _RC, :]
            mx = jnp.max(xp, axis=1, keepdims=True)
            lse = mx + jnp.log(jnp.sum(jnp.exp(xp - mx), axis=1,
                                       keepdims=True))
            ce = -jnp.sum((xp - lse) * g[:, 4:9], axis=1, keepdims=True)
            npd += jnp.sum(ce * vf)
            dch = pch_ref[rb:rb + _RC, :] - g[:, 9:10]
            nc += jnp.sum(dch * dch * vf)
            sx = stop_ref[rb:rb + _RC, :]
            ss += jnp.sum(_softplus_bce(sx, 1.0 - vf))
        acc_ref[2] = nd
        acc_ref[3] = nm
        acc_ref[4] = npd
        acc_ref[5] = nc
        acc_ref[6] = dv
        acc_ref[7] = ss

    x = lg_ref[...]
    hb = hbc_ref[...]
    htp = htp_ref[...]
    inb = (i * _C + lax.broadcasted_iota(jnp.int32, (1, _C), 1)) < _NH
    ohb = (lax.broadcasted_iota(jnp.int32, (_B, _C), 0) == hb)
    ppe_h = lax.dot_general(pperow_ref[0:1, :], ohb.astype(jnp.float32),
                            (((1,), (0,)), ((), ())),
                            preferred_element_type=jnp.float32)
    t_row = lax.broadcasted_iota(jnp.int32, (_T, _C), 0)
    valid = (t_row.astype(jnp.float32) < ppe_h) & inb
    y = (htp == t_row) & valid
    sp = jnp.maximum(x, 0.0) + jnp.log1p(jnp.exp(-jnp.abs(x)))
    contrib = (jnp.sum(jnp.where(valid, sp, 0.0))
               - jnp.sum(jnp.where(y, x, 0.0)))
    dcontrib = jnp.sum(valid.astype(jnp.float32))
    acc_ref[0] += contrib
    acc_ref[1] += dcontrib

    @pl.when(i == _NBLK - 1)
    def _finalize():
        den = jnp.maximum(acc_ref[6], 1.0)
        l_dir = acc_ref[2] / den
        l_mag = acc_ref[3] / den
        l_pid = acc_ref[4] / den
        l_chg = acc_ref[5] / den
        l_asn = acc_ref[0] / jnp.maximum(acc_ref[1], 1.0)
        l_stp = acc_ref[7] / float(_T * _B)
        out_ref[0] = (l_dir + l_mag + l_pid + 0.5 * l_chg + l_asn
                      + 0.5 * l_stp)
        out_ref[1] = l_dir
        out_ref[2] = l_mag
        out_ref[3] = l_pid
        out_ref[4] = l_chg
        out_ref[5] = l_asn
        out_ref[6] = l_stp


def kernel(pfo_momentum, pfo_p_mod, pfo_pid, pfo_charge, assignments,
           assignments_logits, stop_logits, gt_momentum, gt_p_mod, gt_pid,
           gt_charge, gt_batch, hit_to_pfo, hit_batch):
    del assignments
    lg = jnp.swapaxes(assignments_logits, 1, 2)
    hbc = hit_batch.astype(jnp.int32).reshape(1, _NH)
    htp2 = hit_to_pfo.astype(jnp.int32).reshape(1, _NH)
    gtb = gt_batch.astype(jnp.int32)
    gtbr = gtb.reshape(1, _NPFO)
    gtbc = gtb.reshape(_NPFO, 1)
    pmom = pfo_momentum.reshape(_T * _B, 3)
    ppm = pfo_p_mod.reshape(_T * _B, 1)
    ppid = pfo_pid.reshape(_T * _B, 5)
    pch = pfo_charge.reshape(_T * _B, 1)
    stp = stop_logits.reshape(_T * _B, 1)

    full = lambda s: pl.BlockSpec(s, lambda i: (0, 0))
    out = pl.pallas_call(
        _body,
        grid=(_NBLK,),
        in_specs=[
            pl.BlockSpec((_T, None, _C), lambda i: (0, 0, i)),
            pl.BlockSpec((1, _C), lambda i: (0, i)),
            pl.BlockSpec((1, _C), lambda i: (0, i)),
            full((1, _NPFO)),
            full((_NPFO, 1)),
            full((_NPFO, 3)),
            full((_NPFO, 1)),
            full((_NPFO, 5)),
            full((_NPFO, 1)),
            full((_T * _B, 3)),
            full((_T * _B, 1)),
            full((_T * _B, 5)),
            full((_T * _B, 1)),
            full((_T * _B, 1)),
        ],
        out_specs=pl.BlockSpec(memory_space=pltpu.SMEM),
        out_shape=jax.ShapeDtypeStruct((8,), jnp.float32),
        scratch_shapes=[pltpu.SMEM((8,), jnp.float32),
                        pltpu.VMEM((8, _B), jnp.float32)],
        compiler_params=pltpu.CompilerParams(
            dimension_semantics=("arbitrary",)),
    )(lg, hbc, htp2, gtbr, gtbc, gt_momentum, gt_p_mod, gt_pid, gt_charge,
      pmom, ppm, ppid, pch, stp)
    return (out[0], out[1], out[2], out[3], out[4], out[5], out[6])


# speedup vs baseline: 2.5979x; 1.0587x over previous
import jax
import jax.numpy as jnp
from jax import lax
from jax.experimental import pallas as pl
from jax.experimental.pallas import tpu as pltpu

_T, _B, _NPFO, _NH = 32, 64, 1280, 500000
_EPS = 1e-08
_C = 32768
_NBLK = (_NH + _C - 1) // _C
_RC = 256
_JC = 256


def _softplus_bce(x, y):
    return jnp.maximum(x, 0.0) - x * y + jnp.log1p(jnp.exp(-jnp.abs(x)))


def _body(lg_ref, hbc_ref, htp_ref, gtbr_ref, gtbc_ref, gmom_ref, gpm_ref,
          gpid_ref, gch_ref, pmom_ref, ppm_ref, ppid_ref, pch_ref, stop_ref,
          out_ref, acc_ref, pperow_ref):
    i = pl.program_id(0)

    @pl.when(i == 0)
    def _small_terms():
        gtbc = gtbc_ref[...]
        b_iota_r = lax.broadcasted_iota(jnp.int32, (_NPFO, _B), 1)
        ppe_row = jnp.sum((gtbc == b_iota_r).astype(jnp.float32), axis=0,
                          keepdims=True)
        pperow_ref[...] = jnp.broadcast_to(ppe_row, (8, _B))
        acc_ref[0] = 0.0
        acc_ref[1] = 0.0
        gtbr = gtbr_ref[...]
        step_row = jnp.zeros((1, _NPFO), jnp.float32)
        for jc in range(0, _NPFO, _JC):
            gc = gtbc[jc:jc + _JC, :]
            eq = (gc == gtbr)
            jlt = ((jc + lax.broadcasted_iota(jnp.int32, (_JC, _NPFO), 0))
                   < lax.broadcasted_iota(jnp.int32, (_JC, _NPFO), 1))
            step_row = step_row + jnp.sum((eq & jlt).astype(jnp.float32),
                                          axis=0, keepdims=True)
        valid_r = step_row < float(_T)
        c_row = (jnp.minimum(step_row, float(_T - 1)) * float(_B)
                 + gtbr.astype(jnp.float32))

        mom = gmom_ref[...]
        nrm = jnp.sqrt(jnp.sum(mom * mom, axis=1, keepdims=True))
        gdir = mom / jnp.maximum(nrm, _EPS)
        x5 = gpid_ref[...]
        m5 = jnp.max(x5, axis=1, keepdims=True)
        li = lax.broadcasted_iota(jnp.int32, (_NPFO, 5), 1)
        idx5 = jnp.min(jnp.where(x5 == m5, li, 5), axis=1, keepdims=True)
        oh5 = (li == idx5).astype(jnp.float32)
        packed = jnp.concatenate(
            [gdir, gpm_ref[...], oh5, gch_ref[...]], axis=1)

        ppe_col = jnp.sum(
            (lax.broadcasted_iota(jnp.int32, (_B, _NPFO), 0) == gtbr)
            .astype(jnp.float32), axis=1, keepdims=True)

        nd = nm = npd = nc = dv = ss = 0.0
        for rb in range(0, _T * _B, _RC):
            rid0 = (rb + lax.broadcasted_iota(jnp.int32, (_RC, _NPFO), 0))
            mm = ((rid0.astype(jnp.float32) == c_row) & valid_r)
            g = lax.dot_general(mm.astype(jnp.float32), packed,
                                (((1,), (0,)), ((), ())),
                                preferred_element_type=jnp.float32)
            rid1 = rb + lax.broadcasted_iota(jnp.int32, (_RC, 1), 0)
            t_c = lax.shift_right_logical(rid1, 6).astype(jnp.float32)
            ohb2 = (lax.broadcasted_iota(jnp.int32, (_RC, _B), 1)
                    == jnp.bitwise_and(rid1, _B - 1)).astype(jnp.float32)
            ppe_c = lax.dot_general(ohb2, ppe_col, (((1,), (0,)), ((), ())),
                                    preferred_element_type=jnp.float32)
            vf = (t_c < ppe_c).astype(jnp.float32)
            dv += jnp.sum(vf)
            pm = pmom_ref[rb:rb + _RC, :]
            pn = jnp.sqrt(jnp.sum(pm * pm, axis=1, keepdims=True))
            pdir = pm / jnp.maximum(pn, _EPS)
            cos = jnp.sum(pdir * g[:, 0:3], axis=1, keepdims=True)
            nd += jnp.sum((1.0 - cos) * vf)
            dpm = ppm_ref[rb:rb + _RC, :] - g[:, 3:4]
            nm += jnp.sum(dpm * dpm * vf)
            xp = ppid_ref[rb:rb + _RC, :]
            mx = jnp.max(xp, axis=1, keepdims=True)
            lse = mx + jnp.log(jnp.sum(jnp.exp(xp - mx), axis=1,
                                       keepdims=True))
            ce = -jnp.sum((xp - lse) * g[:, 4:9], axis=1, keepdims=True)
            npd += jnp.sum(ce * vf)
            dch = pch_ref[rb:rb + _RC, :] - g[:, 9:10]
            nc += jnp.sum(dch * dch * vf)
            sx = stop_ref[rb:rb + _RC, :]
            ss += jnp.sum(_softplus_bce(sx, 1.0 - vf))
        acc_ref[2] = nd
        acc_ref[3] = nm
        acc_ref[4] = npd
        acc_ref[5] = nc
        acc_ref[6] = dv
        acc_ref[7] = ss

    x = lg_ref[...]
    hb = hbc_ref[...]
    htp = htp_ref[...]
    inb = (i * _C + lax.broadcasted_iota(jnp.int32, (1, _C), 1)) < _NH
    ohb = (lax.broadcasted_iota(jnp.int32, (_B, _C), 0) == hb)
    ppe_h = lax.dot_general(pperow_ref[0:1, :], ohb.astype(jnp.float32),
                            (((1,), (0,)), ((), ())),
                            preferred_element_type=jnp.float32)
    t_row = lax.broadcasted_iota(jnp.int32, (_T, _C), 0)
    valid = (t_row.astype(jnp.float32) < ppe_h) & inb
    y = (htp == t_row) & valid
    sp = jnp.maximum(x, 0.0) + jnp.log1p(jnp.exp(-jnp.abs(x)))
    contrib = (jnp.sum(jnp.where(valid, sp, 0.0))
               - jnp.sum(jnp.where(y, x, 0.0)))
    dcontrib = jnp.sum(valid.astype(jnp.float32))
    acc_ref[0] += contrib
    acc_ref[1] += dcontrib

    @pl.when(i == _NBLK - 1)
    def _finalize():
        den = jnp.maximum(acc_ref[6], 1.0)
        l_dir = acc_ref[2] / den
        l_mag = acc_ref[3] / den
        l_pid = acc_ref[4] / den
        l_chg = acc_ref[5] / den
        l_asn = acc_ref[0] / jnp.maximum(acc_ref[1], 1.0)
        l_stp = acc_ref[7] / float(_T * _B)
        out_ref[0] = (l_dir + l_mag + l_pid + 0.5 * l_chg + l_asn
                      + 0.5 * l_stp)
        out_ref[1] = l_dir
        out_ref[2] = l_mag
        out_ref[3] = l_pid
        out_ref[4] = l_chg
        out_ref[5] = l_asn
        out_ref[6] = l_stp


def kernel(pfo_momentum, pfo_p_mod, pfo_pid, pfo_charge, assignments,
           assignments_logits, stop_logits, gt_momentum, gt_p_mod, gt_pid,
           gt_charge, gt_batch, hit_to_pfo, hit_batch):
    del assignments
    lg = jnp.swapaxes(assignments_logits, 1, 2)
    hbc = hit_batch.astype(jnp.int32).reshape(1, _NH)
    htp2 = hit_to_pfo.astype(jnp.int32).reshape(1, _NH)
    gtb = gt_batch.astype(jnp.int32)
    gtbr = gtb.reshape(1, _NPFO)
    gtbc = gtb.reshape(_NPFO, 1)
    pmom = pfo_momentum.reshape(_T * _B, 3)
    ppm = pfo_p_mod.reshape(_T * _B, 1)
    ppid = pfo_pid.reshape(_T * _B, 5)
    pch = pfo_charge.reshape(_T * _B, 1)
    stp = stop_logits.reshape(_T * _B, 1)

    full = lambda s: pl.BlockSpec(s, lambda i: (0, 0))
    out = pl.pallas_call(
        _body,
        grid=(_NBLK,),
        in_specs=[
            pl.BlockSpec((_T, None, _C), lambda i: (0, 0, i)),
            pl.BlockSpec((1, _C), lambda i: (0, i)),
            pl.BlockSpec((1, _C), lambda i: (0, i)),
            full((1, _NPFO)),
            full((_NPFO, 1)),
            full((_NPFO, 3)),
            full((_NPFO, 1)),
            full((_NPFO, 5)),
            full((_NPFO, 1)),
            full((_T * _B, 3)),
            full((_T * _B, 1)),
            full((_T * _B, 5)),
            full((_T * _B, 1)),
            full((_T * _B, 1)),
        ],
        out_specs=pl.BlockSpec(memory_space=pltpu.SMEM),
        out_shape=jax.ShapeDtypeStruct((8,), jnp.float32),
        scratch_shapes=[pltpu.SMEM((8,), jnp.float32),
                        pltpu.VMEM((8, _B), jnp.float32)],
        compiler_params=pltpu.CompilerParams(
            dimension_semantics=("arbitrary",)),
    )(lg, hbc, htp2, gtbr, gtbc, gt_momentum, gt_p_mod, gt_pid, gt_charge,
      pmom, ppm, ppid, pch, stp)
    return (out[0], out[1], out[2], out[3], out[4], out[5], out[6])


# speedup vs baseline: 3.2372x; 1.2461x over previous
import jax
import jax.numpy as jnp
from jax import lax
from jax.experimental import pallas as pl
from jax.experimental.pallas import tpu as pltpu

_T, _B, _NPFO, _NH = 32, 64, 1280, 500000
_EPS = 1e-08
_C = 32768
_NBLK = (_NH + _C - 1) // _C
_RC = 256
_JC = 256


def _softplus_bce(x, y):
    return jnp.maximum(x, 0.0) - x * y + jnp.log1p(jnp.exp(-jnp.abs(x)))


_TAIL = _NH - (_NBLK - 1) * _C


def _copy_block(lg_hbm, vbuf_ref, sem, j, slot):
    return pltpu.make_async_copy(
        lg_hbm.at[:, 0, pl.ds(j * _C, _C)],
        vbuf_ref.at[slot], sem.at[slot])


def _copy_tail(lg_hbm, tail_ref, sem, slot):
    return pltpu.make_async_copy(
        lg_hbm.at[:, 0, pl.ds((_NBLK - 1) * _C, _TAIL)],
        tail_ref, sem.at[slot])


def _assign_block(x, hb, htp, ppe_row):
    w = x.shape[1]
    ohb = (lax.broadcasted_iota(jnp.int32, (_B, w), 0) == hb)
    ppe_h = lax.dot_general(ppe_row, ohb.astype(jnp.float32),
                            (((1,), (0,)), ((), ())),
                            preferred_element_type=jnp.float32)
    t_row = lax.broadcasted_iota(jnp.int32, (_T, w), 0)
    valid = t_row.astype(jnp.float32) < ppe_h
    y = (htp == t_row) & valid
    sp = jnp.maximum(x, 0.0) + jnp.log1p(jnp.exp(-jnp.abs(x)))
    contrib = (jnp.sum(jnp.where(valid, sp, 0.0))
               - jnp.sum(jnp.where(y, x, 0.0)))
    dcontrib = jnp.sum(valid.astype(jnp.float32))
    return contrib, dcontrib


def _body(lg_hbm, hbc_ref, htp_ref, gtbr_ref, gtbc_ref, gmom_ref, gpm_ref,
          gpid_ref, gch_ref, pmom_ref, ppm_ref, ppid_ref, pch_ref, stop_ref,
          out_ref, acc_ref, pperow_ref, vbuf_ref, tail_ref, sem):
    i = pl.program_id(0)
    slot = lax.rem(i, 2)
    nslot = lax.rem(i + 1, 2)

    @pl.when(i == 0)
    def _first_copy():
        _copy_block(lg_hbm, vbuf_ref, sem, i, slot).start()

    @pl.when(i + 1 < _NBLK - 1)
    def _prefetch_full():
        _copy_block(lg_hbm, vbuf_ref, sem, i + 1, nslot).start()

    @pl.when(i + 1 == _NBLK - 1)
    def _prefetch_tail():
        _copy_tail(lg_hbm, tail_ref, sem, nslot).start()

    @pl.when(i == 0)
    def _small_terms():
        gtbc = gtbc_ref[...]
        b_iota_r = lax.broadcasted_iota(jnp.int32, (_NPFO, _B), 1)
        ppe_row = jnp.sum((gtbc == b_iota_r).astype(jnp.float32), axis=0,
                          keepdims=True)
        pperow_ref[...] = jnp.broadcast_to(ppe_row, (8, _B))
        acc_ref[0] = 0.0
        acc_ref[1] = 0.0
        gtbr = gtbr_ref[...]
        step_row = jnp.zeros((1, _NPFO), jnp.float32)
        for jc in range(0, _NPFO, _JC):
            gc = gtbc[jc:jc + _JC, :]
            eq = (gc == gtbr)
            jlt = ((jc + lax.broadcasted_iota(jnp.int32, (_JC, _NPFO), 0))
                   < lax.broadcasted_iota(jnp.int32, (_JC, _NPFO), 1))
            step_row = step_row + jnp.sum((eq & jlt).astype(jnp.float32),
                                          axis=0, keepdims=True)
        valid_r = step_row < float(_T)
        c_row = (jnp.minimum(step_row, float(_T - 1)) * float(_B)
                 + gtbr.astype(jnp.float32))

        mom = gmom_ref[...]
        nrm = jnp.sqrt(jnp.sum(mom * mom, axis=1, keepdims=True))
        gdir = mom / jnp.maximum(nrm, _EPS)
        x5 = gpid_ref[...]
        m5 = jnp.max(x5, axis=1, keepdims=True)
        li = lax.broadcasted_iota(jnp.int32, (_NPFO, 5), 1)
        idx5 = jnp.min(jnp.where(x5 == m5, li, 5), axis=1, keepdims=True)
        oh5 = (li == idx5).astype(jnp.float32)
        packed = jnp.concatenate(
            [gdir, gpm_ref[...], oh5, gch_ref[...]], axis=1)

        ppe_col = jnp.sum(
            (lax.broadcasted_iota(jnp.int32, (_B, _NPFO), 0) == gtbr)
            .astype(jnp.float32), axis=1, keepdims=True)

        nd = nm = npd = nc = dv = ss = 0.0
        for rb in range(0, _T * _B, _RC):
            rid0 = (rb + lax.broadcasted_iota(jnp.int32, (_RC, _NPFO), 0))
            mm = ((rid0.astype(jnp.float32) == c_row) & valid_r)
            g = lax.dot_general(mm.astype(jnp.float32), packed,
                                (((1,), (0,)), ((), ())),
                                preferred_element_type=jnp.float32)
            rid1 = rb + lax.broadcasted_iota(jnp.int32, (_RC, 1), 0)
            t_c = lax.shift_right_logical(rid1, 6).astype(jnp.float32)
            ohb2 = (lax.broadcasted_iota(jnp.int32, (_RC, _B), 1)
                    == jnp.bitwise_and(rid1, _B - 1)).astype(jnp.float32)
            ppe_c = lax.dot_general(ohb2, ppe_col, (((1,), (0,)), ((), ())),
                                    preferred_element_type=jnp.float32)
            vf = (t_c < ppe_c).astype(jnp.float32)
            dv += jnp.sum(vf)
            pm = pmom_ref[rb:rb + _RC, :]
            pn = jnp.sqrt(jnp.sum(pm * pm, axis=1, keepdims=True))
            pdir = pm / jnp.maximum(pn, _EPS)
            cos = jnp.sum(pdir * g[:, 0:3], axis=1, keepdims=True)
            nd += jnp.sum((1.0 - cos) * vf)
            dpm = ppm_ref[rb:rb + _RC, :] - g[:, 3:4]
            nm += jnp.sum(dpm * dpm * vf)
            xp = ppid_ref[rb:rb + _RC, :]
            mx = jnp.max(xp, axis=1, keepdims=True)
            lse = mx + jnp.log(jnp.sum(jnp.exp(xp - mx), axis=1,
                                       keepdims=True))
            ce = -jnp.sum((xp - lse) * g[:, 4:9], axis=1, keepdims=True)
            npd += jnp.sum(ce * vf)
            dch = pch_ref[rb:rb + _RC, :] - g[:, 9:10]
            nc += jnp.sum(dch * dch * vf)
            sx = stop_ref[rb:rb + _RC, :]
            ss += jnp.sum(_softplus_bce(sx, 1.0 - vf))
        acc_ref[2] = nd
        acc_ref[3] = nm
        acc_ref[4] = npd
        acc_ref[5] = nc
        acc_ref[6] = dv
        acc_ref[7] = ss

    @pl.when(i < _NBLK - 1)
    def _full_block():
        _copy_block(lg_hbm, vbuf_ref, sem, i, slot).wait()
        c, d = _assign_block(vbuf_ref[slot], hbc_ref[...], htp_ref[...],
                             pperow_ref[0:1, :])
        acc_ref[0] += c
        acc_ref[1] += d

    @pl.when(i == _NBLK - 1)
    def _tail_block():
        _copy_tail(lg_hbm, tail_ref, sem, slot).wait()
        c, d = _assign_block(tail_ref[...], hbc_ref[:, :_TAIL],
                             htp_ref[:, :_TAIL], pperow_ref[0:1, :])
        acc_ref[0] += c
        acc_ref[1] += d

    @pl.when(i == _NBLK - 1)
    def _finalize():
        den = jnp.maximum(acc_ref[6], 1.0)
        l_dir = acc_ref[2] / den
        l_mag = acc_ref[3] / den
        l_pid = acc_ref[4] / den
        l_chg = acc_ref[5] / den
        l_asn = acc_ref[0] / jnp.maximum(acc_ref[1], 1.0)
        l_stp = acc_ref[7] / float(_T * _B)
        out_ref[0] = (l_dir + l_mag + l_pid + 0.5 * l_chg + l_asn
                      + 0.5 * l_stp)
        out_ref[1] = l_dir
        out_ref[2] = l_mag
        out_ref[3] = l_pid
        out_ref[4] = l_chg
        out_ref[5] = l_asn
        out_ref[6] = l_stp


def kernel(pfo_momentum, pfo_p_mod, pfo_pid, pfo_charge, assignments,
           assignments_logits, stop_logits, gt_momentum, gt_p_mod, gt_pid,
           gt_charge, gt_batch, hit_to_pfo, hit_batch):
    del assignments
    lg = jnp.swapaxes(assignments_logits, 1, 2)
    hbc = hit_batch.astype(jnp.int32).reshape(1, _NH)
    htp2 = hit_to_pfo.astype(jnp.int32).reshape(1, _NH)
    gtb = gt_batch.astype(jnp.int32)
    gtbr = gtb.reshape(1, _NPFO)
    gtbc = gtb.reshape(_NPFO, 1)
    pmom = pfo_momentum.reshape(_T * _B, 3)
    ppm = pfo_p_mod.reshape(_T * _B, 1)
    ppid = pfo_pid.reshape(_T * _B, 5)
    pch = pfo_charge.reshape(_T * _B, 1)
    stp = stop_logits.reshape(_T * _B, 1)

    full = lambda s: pl.BlockSpec(s, lambda i: (0, 0))
    out = pl.pallas_call(
        _body,
        grid=(_NBLK,),
        in_specs=[
            pl.BlockSpec(memory_space=pltpu.MemorySpace.HBM),
            pl.BlockSpec((1, _C), lambda i: (0, i)),
            pl.BlockSpec((1, _C), lambda i: (0, i)),
            full((1, _NPFO)),
            full((_NPFO, 1)),
            full((_NPFO, 3)),
            full((_NPFO, 1)),
            full((_NPFO, 5)),
            full((_NPFO, 1)),
            full((_T * _B, 3)),
            full((_T * _B, 1)),
            full((_T * _B, 5)),
            full((_T * _B, 1)),
            full((_T * _B, 1)),
        ],
        out_specs=pl.BlockSpec(memory_space=pltpu.SMEM),
        out_shape=jax.ShapeDtypeStruct((8,), jnp.float32),
        scratch_shapes=[pltpu.SMEM((8,), jnp.float32),
                        pltpu.VMEM((8, _B), jnp.float32),
                        pltpu.VMEM((2, _T, _C), jnp.float32),
                        pltpu.VMEM((_T, _TAIL), jnp.float32),
                        pltpu.SemaphoreType.DMA((2,))],
        compiler_params=pltpu.CompilerParams(
            dimension_semantics=("arbitrary",)),
    )(lg, hbc, htp2, gtbr, gtbc, gt_momentum, gt_p_mod, gt_pid, gt_charge,
      pmom, ppm, ppid, pch, stp)
    return (out[0], out[1], out[2], out[3], out[4], out[5], out[6])


# speedup vs baseline: 3.3491x; 1.0346x over previous
import jax
import jax.numpy as jnp
from jax import lax
from jax.experimental import pallas as pl
from jax.experimental.pallas import tpu as pltpu

_T, _B, _NPFO, _NH = 32, 64, 1280, 500000
_EPS = 1e-08
_C = 32768
_NBLK = (_NH + _C - 1) // _C
_RC = 256
_JC = 256


def _softplus_bce(x, y):
    return jnp.maximum(x, 0.0) - x * y + jnp.log1p(jnp.exp(-jnp.abs(x)))


_TAIL = _NH - (_NBLK - 1) * _C


def _copy_block(lg_hbm, vbuf_ref, sem, j, slot):
    return pltpu.make_async_copy(
        lg_hbm.at[:, 0, pl.ds(j * _C, _C)],
        vbuf_ref.at[slot], sem.at[slot])


def _copy_tail(lg_hbm, tail_ref, sem, slot):
    return pltpu.make_async_copy(
        lg_hbm.at[:, 0, pl.ds((_NBLK - 1) * _C, _TAIL)],
        tail_ref, sem.at[slot])


def _assign_block(x, hb, htp, ppe_row):
    w = x.shape[1]
    ohb = (lax.broadcasted_iota(jnp.int32, (_B, w), 0) == hb)
    ppe_h = lax.dot_general(ppe_row, ohb.astype(jnp.float32),
                            (((1,), (0,)), ((), ())),
                            preferred_element_type=jnp.float32)
    t_row = lax.broadcasted_iota(jnp.int32, (_T, w), 0)
    valid = t_row.astype(jnp.float32) < ppe_h
    sp = jnp.maximum(x, 0.0) + jnp.log1p(jnp.exp(-jnp.abs(x)))
    wv = jnp.where(htp == t_row, sp - x, sp)
    contrib = jnp.sum(jnp.where(valid, wv, 0.0))
    dcontrib = jnp.sum(jnp.minimum(ppe_h, float(_T)))
    return contrib, dcontrib


def _body(lg_hbm, hbc_ref, htp_ref, gtbr_ref, gtbc_ref, gmom_ref, gpm_ref,
          gpid_ref, gch_ref, pmom_ref, ppm_ref, ppid_ref, pch_ref, stop_ref,
          out_ref, acc_ref, pperow_ref, vbuf_ref, tail_ref, sem):
    i = pl.program_id(0)
    slot = lax.rem(i, 2)
    nslot = lax.rem(i + 1, 2)

    @pl.when(i == 0)
    def _first_copy():
        _copy_block(lg_hbm, vbuf_ref, sem, i, slot).start()

    @pl.when(i + 1 < _NBLK - 1)
    def _prefetch_full():
        _copy_block(lg_hbm, vbuf_ref, sem, i + 1, nslot).start()

    @pl.when(i + 1 == _NBLK - 1)
    def _prefetch_tail():
        _copy_tail(lg_hbm, tail_ref, sem, nslot).start()

    @pl.when(i == 0)
    def _small_terms():
        gtbc = gtbc_ref[...]
        b_iota_r = lax.broadcasted_iota(jnp.int32, (_NPFO, _B), 1)
        ppe_row = jnp.sum((gtbc == b_iota_r).astype(jnp.float32), axis=0,
                          keepdims=True)
        pperow_ref[...] = jnp.broadcast_to(ppe_row, (8, _B))
        acc_ref[0] = 0.0
        acc_ref[1] = 0.0
        gtbr = gtbr_ref[...]
        step_row = jnp.zeros((1, _NPFO), jnp.float32)
        for jc in range(0, _NPFO, _JC):
            gc = gtbc[jc:jc + _JC, :]
            eq = (gc == gtbr)
            jlt = ((jc + lax.broadcasted_iota(jnp.int32, (_JC, _NPFO), 0))
                   < lax.broadcasted_iota(jnp.int32, (_JC, _NPFO), 1))
            step_row = step_row + jnp.sum((eq & jlt).astype(jnp.float32),
                                          axis=0, keepdims=True)
        valid_r = step_row < float(_T)
        c_row = (jnp.minimum(step_row, float(_T - 1)) * float(_B)
                 + gtbr.astype(jnp.float32))

        mom = gmom_ref[...]
        nrm = jnp.sqrt(jnp.sum(mom * mom, axis=1, keepdims=True))
        gdir = mom / jnp.maximum(nrm, _EPS)
        x5 = gpid_ref[...]
        m5 = jnp.max(x5, axis=1, keepdims=True)
        li = lax.broadcasted_iota(jnp.int32, (_NPFO, 5), 1)
        idx5 = jnp.min(jnp.where(x5 == m5, li, 5), axis=1, keepdims=True)
        oh5 = (li == idx5).astype(jnp.float32)
        packed = jnp.concatenate(
            [gdir, gpm_ref[...], oh5, gch_ref[...]], axis=1)

        ppe_col = jnp.sum(
            (lax.broadcasted_iota(jnp.int32, (_B, _NPFO), 0) == gtbr)
            .astype(jnp.float32), axis=1, keepdims=True)

        nd = nm = npd = nc = dv = ss = 0.0
        for rb in range(0, _T * _B, _RC):
            rid0 = (rb + lax.broadcasted_iota(jnp.int32, (_RC, _NPFO), 0))
            mm = ((rid0.astype(jnp.float32) == c_row) & valid_r)
            g = lax.dot_general(mm.astype(jnp.float32), packed,
                                (((1,), (0,)), ((), ())),
                                preferred_element_type=jnp.float32)
            rid1 = rb + lax.broadcasted_iota(jnp.int32, (_RC, 1), 0)
            t_c = lax.shift_right_logical(rid1, 6).astype(jnp.float32)
            ohb2 = (lax.broadcasted_iota(jnp.int32, (_RC, _B), 1)
                    == jnp.bitwise_and(rid1, _B - 1)).astype(jnp.float32)
            ppe_c = lax.dot_general(ohb2, ppe_col, (((1,), (0,)), ((), ())),
                                    preferred_element_type=jnp.float32)
            vf = (t_c < ppe_c).astype(jnp.float32)
            dv += jnp.sum(vf)
            pm = pmom_ref[rb:rb + _RC, :]
            pn = jnp.sqrt(jnp.sum(pm * pm, axis=1, keepdims=True))
            pdir = pm / jnp.maximum(pn, _EPS)
            cos = jnp.sum(pdir * g[:, 0:3], axis=1, keepdims=True)
            nd += jnp.sum((1.0 - cos) * vf)
            dpm = ppm_ref[rb:rb + _RC, :] - g[:, 3:4]
            nm += jnp.sum(dpm * dpm * vf)
            xp = ppid_ref[rb:rb + _RC, :]
            mx = jnp.max(xp, axis=1, keepdims=True)
            lse = mx + jnp.log(jnp.sum(jnp.exp(xp - mx), axis=1,
                                       keepdims=True))
            ce = -jnp.sum((xp - lse) * g[:, 4:9], axis=1, keepdims=True)
            npd += jnp.sum(ce * vf)
            dch = pch_ref[rb:rb + _RC, :] - g[:, 9:10]
            nc += jnp.sum(dch * dch * vf)
            sx = stop_ref[rb:rb + _RC, :]
            ss += jnp.sum(_softplus_bce(sx, 1.0 - vf))
        acc_ref[2] = nd
        acc_ref[3] = nm
        acc_ref[4] = npd
        acc_ref[5] = nc
        acc_ref[6] = dv
        acc_ref[7] = ss

    @pl.when(i < _NBLK - 1)
    def _full_block():
        _copy_block(lg_hbm, vbuf_ref, sem, i, slot).wait()
        c, d = _assign_block(vbuf_ref[slot], hbc_ref[...], htp_ref[...],
                             pperow_ref[0:1, :])
        acc_ref[0] += c
        acc_ref[1] += d

    @pl.when(i == _NBLK - 1)
    def _tail_block():
        _copy_tail(lg_hbm, tail_ref, sem, slot).wait()
        c, d = _assign_block(tail_ref[...], hbc_ref[:, :_TAIL],
                             htp_ref[:, :_TAIL], pperow_ref[0:1, :])
        acc_ref[0] += c
        acc_ref[1] += d

    @pl.when(i == _NBLK - 1)
    def _finalize():
        den = jnp.maximum(acc_ref[6], 1.0)
        l_dir = acc_ref[2] / den
        l_mag = acc_ref[3] / den
        l_pid = acc_ref[4] / den
        l_chg = acc_ref[5] / den
        l_asn = acc_ref[0] / jnp.maximum(acc_ref[1], 1.0)
        l_stp = acc_ref[7] / float(_T * _B)
        out_ref[0] = (l_dir + l_mag + l_pid + 0.5 * l_chg + l_asn
                      + 0.5 * l_stp)
        out_ref[1] = l_dir
        out_ref[2] = l_mag
        out_ref[3] = l_pid
        out_ref[4] = l_chg
        out_ref[5] = l_asn
        out_ref[6] = l_stp


def kernel(pfo_momentum, pfo_p_mod, pfo_pid, pfo_charge, assignments,
           assignments_logits, stop_logits, gt_momentum, gt_p_mod, gt_pid,
           gt_charge, gt_batch, hit_to_pfo, hit_batch):
    del assignments
    lg = jnp.swapaxes(assignments_logits, 1, 2)
    hbc = hit_batch.astype(jnp.int32).reshape(1, _NH)
    htp2 = hit_to_pfo.astype(jnp.int32).reshape(1, _NH)
    gtb = gt_batch.astype(jnp.int32)
    gtbr = gtb.reshape(1, _NPFO)
    gtbc = gtb.reshape(_NPFO, 1)
    pmom = pfo_momentum.reshape(_T * _B, 3)
    ppm = pfo_p_mod.reshape(_T * _B, 1)
    ppid = pfo_pid.reshape(_T * _B, 5)
    pch = pfo_charge.reshape(_T * _B, 1)
    stp = stop_logits.reshape(_T * _B, 1)

    full = lambda s: pl.BlockSpec(s, lambda i: (0, 0))
    out = pl.pallas_call(
        _body,
        grid=(_NBLK,),
        in_specs=[
            pl.BlockSpec(memory_space=pltpu.MemorySpace.HBM),
            pl.BlockSpec((1, _C), lambda i: (0, i)),
            pl.BlockSpec((1, _C), lambda i: (0, i)),
            full((1, _NPFO)),
            full((_NPFO, 1)),
            full((_NPFO, 3)),
            full((_NPFO, 1)),
            full((_NPFO, 5)),
            full((_NPFO, 1)),
            full((_T * _B, 3)),
            full((_T * _B, 1)),
            full((_T * _B, 5)),
            full((_T * _B, 1)),
            full((_T * _B, 1)),
        ],
        out_specs=pl.BlockSpec(memory_space=pltpu.SMEM),
        out_shape=jax.ShapeDtypeStruct((8,), jnp.float32),
        scratch_shapes=[pltpu.SMEM((8,), jnp.float32),
                        pltpu.VMEM((8, _B), jnp.float32),
                        pltpu.VMEM((2, _T, _C), jnp.float32),
                        pltpu.VMEM((_T, _TAIL), jnp.float32),
                        pltpu.SemaphoreType.DMA((2,))],
        compiler_params=pltpu.CompilerParams(
            dimension_semantics=("arbitrary",)),
    )(lg, hbc, htp2, gtbr, gtbc, gt_momentum, gt_p_mod, gt_pid, gt_charge,
      pmom, ppm, ppid, pch, stp)
    return (out[0], out[1], out[2], out[3], out[4], out[5], out[6])


# speedup vs baseline: 3.4104x; 1.0183x over previous
import jax
import jax.numpy as jnp
from jax import lax
from jax.experimental import pallas as pl
from jax.experimental.pallas import tpu as pltpu

_T, _B, _NPFO, _NH = 32, 64, 1280, 500000
_EPS = 1e-08
_C = 65536
_NBLK = (_NH + _C - 1) // _C
_RC = 256
_JC = 256


def _softplus_bce(x, y):
    return jnp.maximum(x, 0.0) - x * y + jnp.log1p(jnp.exp(-jnp.abs(x)))


_TAIL = _NH - (_NBLK - 1) * _C


def _copy_block(lg_hbm, vbuf_ref, sem, j, slot):
    return pltpu.make_async_copy(
        lg_hbm.at[:, 0, pl.ds(j * _C, _C)],
        vbuf_ref.at[slot], sem.at[slot])


def _copy_tail(lg_hbm, tail_ref, sem, slot):
    return pltpu.make_async_copy(
        lg_hbm.at[:, 0, pl.ds((_NBLK - 1) * _C, _TAIL)],
        tail_ref, sem.at[slot])


def _assign_block(x, hb, htp, ppe_row):
    w = x.shape[1]
    ohb = (lax.broadcasted_iota(jnp.int32, (_B, w), 0) == hb)
    ppe_h = lax.dot_general(ppe_row, ohb.astype(jnp.float32),
                            (((1,), (0,)), ((), ())),
                            preferred_element_type=jnp.float32)
    t_row = lax.broadcasted_iota(jnp.int32, (_T, w), 0)
    valid = t_row.astype(jnp.float32) < ppe_h
    sp = jnp.maximum(x, 0.0) + jnp.log1p(jnp.exp(-jnp.abs(x)))
    wv = jnp.where(htp == t_row, sp - x, sp)
    contrib = jnp.sum(jnp.where(valid, wv, 0.0))
    dcontrib = jnp.sum(jnp.minimum(ppe_h, float(_T)))
    return contrib, dcontrib


def _body(lg_hbm, hbc_ref, htp_ref, gtbr_ref, gtbc_ref, gmom_ref, gpm_ref,
          gpid_ref, gch_ref, pmom_ref, ppm_ref, ppid_ref, pch_ref, stop_ref,
          out_ref, acc_ref, pperow_ref, vbuf_ref, tail_ref, sem):
    i = pl.program_id(0)
    slot = lax.rem(i, 2)
    nslot = lax.rem(i + 1, 2)

    @pl.when(i == 0)
    def _first_copy():
        _copy_block(lg_hbm, vbuf_ref, sem, i, slot).start()

    @pl.when(i + 1 < _NBLK - 1)
    def _prefetch_full():
        _copy_block(lg_hbm, vbuf_ref, sem, i + 1, nslot).start()

    @pl.when(i + 1 == _NBLK - 1)
    def _prefetch_tail():
        _copy_tail(lg_hbm, tail_ref, sem, nslot).start()

    @pl.when(i == 0)
    def _small_terms():
        gtbc = gtbc_ref[...]
        b_iota_r = lax.broadcasted_iota(jnp.int32, (_NPFO, _B), 1)
        ppe_row = jnp.sum((gtbc == b_iota_r).astype(jnp.float32), axis=0,
                          keepdims=True)
        pperow_ref[...] = jnp.broadcast_to(ppe_row, (8, _B))
        acc_ref[0] = 0.0
        acc_ref[1] = 0.0
        gtbr = gtbr_ref[...]
        step_row = jnp.zeros((1, _NPFO), jnp.float32)
        for jc in range(0, _NPFO, _JC):
            gc = gtbc[jc:jc + _JC, :]
            eq = (gc == gtbr)
            jlt = ((jc + lax.broadcasted_iota(jnp.int32, (_JC, _NPFO), 0))
                   < lax.broadcasted_iota(jnp.int32, (_JC, _NPFO), 1))
            step_row = step_row + jnp.sum((eq & jlt).astype(jnp.float32),
                                          axis=0, keepdims=True)
        valid_r = step_row < float(_T)
        c_row = (jnp.minimum(step_row, float(_T - 1)) * float(_B)
                 + gtbr.astype(jnp.float32))

        mom = gmom_ref[...]
        nrm = jnp.sqrt(jnp.sum(mom * mom, axis=1, keepdims=True))
        gdir = mom / jnp.maximum(nrm, _EPS)
        x5 = gpid_ref[...]
        m5 = jnp.max(x5, axis=1, keepdims=True)
        li = lax.broadcasted_iota(jnp.int32, (_NPFO, 5), 1)
        idx5 = jnp.min(jnp.where(x5 == m5, li, 5), axis=1, keepdims=True)
        oh5 = (li == idx5).astype(jnp.float32)
        packed = jnp.concatenate(
            [gdir, gpm_ref[...], oh5, gch_ref[...]], axis=1)

        ppe_col = jnp.sum(
            (lax.broadcasted_iota(jnp.int32, (_B, _NPFO), 0) == gtbr)
            .astype(jnp.float32), axis=1, keepdims=True)

        nd = nm = npd = nc = dv = ss = 0.0
        for rb in range(0, _T * _B, _RC):
            rid0 = (rb + lax.broadcasted_iota(jnp.int32, (_RC, _NPFO), 0))
            mm = ((rid0.astype(jnp.float32) == c_row) & valid_r)
            g = lax.dot_general(mm.astype(jnp.float32), packed,
                                (((1,), (0,)), ((), ())),
                                preferred_element_type=jnp.float32)
            rid1 = rb + lax.broadcasted_iota(jnp.int32, (_RC, 1), 0)
            t_c = lax.shift_right_logical(rid1, 6).astype(jnp.float32)
            ohb2 = (lax.broadcasted_iota(jnp.int32, (_RC, _B), 1)
                    == jnp.bitwise_and(rid1, _B - 1)).astype(jnp.float32)
            ppe_c = lax.dot_general(ohb2, ppe_col, (((1,), (0,)), ((), ())),
                                    preferred_element_type=jnp.float32)
            vf = (t_c < ppe_c).astype(jnp.float32)
            dv += jnp.sum(vf)
            pm = pmom_ref[rb:rb + _RC, :]
            pn = jnp.sqrt(jnp.sum(pm * pm, axis=1, keepdims=True))
            pdir = pm / jnp.maximum(pn, _EPS)
            cos = jnp.sum(pdir * g[:, 0:3], axis=1, keepdims=True)
            nd += jnp.sum((1.0 - cos) * vf)
            dpm = ppm_ref[rb:rb + _RC, :] - g[:, 3:4]
            nm += jnp.sum(dpm * dpm * vf)
            xp = ppid_ref[rb:rb + _RC, :]
            mx = jnp.max(xp, axis=1, keepdims=True)
            lse = mx + jnp.log(jnp.sum(jnp.exp(xp - mx), axis=1,
                                       keepdims=True))
            ce = -jnp.sum((xp - lse) * g[:, 4:9], axis=1, keepdims=True)
            npd += jnp.sum(ce * vf)
            dch = pch_ref[rb:rb + _RC, :] - g[:, 9:10]
            nc += jnp.sum(dch * dch * vf)
            sx = stop_ref[rb:rb + _RC, :]
            ss += jnp.sum(_softplus_bce(sx, 1.0 - vf))
        acc_ref[2] = nd
        acc_ref[3] = nm
        acc_ref[4] = npd
        acc_ref[5] = nc
        acc_ref[6] = dv
        acc_ref[7] = ss

    @pl.when(i < _NBLK - 1)
    def _full_block():
        _copy_block(lg_hbm, vbuf_ref, sem, i, slot).wait()
        c, d = _assign_block(vbuf_ref[slot], hbc_ref[...], htp_ref[...],
                             pperow_ref[0:1, :])
        acc_ref[0] += c
        acc_ref[1] += d

    @pl.when(i == _NBLK - 1)
    def _tail_block():
        _copy_tail(lg_hbm, tail_ref, sem, slot).wait()
        c, d = _assign_block(tail_ref[...], hbc_ref[:, :_TAIL],
                             htp_ref[:, :_TAIL], pperow_ref[0:1, :])
        acc_ref[0] += c
        acc_ref[1] += d

    @pl.when(i == _NBLK - 1)
    def _finalize():
        den = jnp.maximum(acc_ref[6], 1.0)
        l_dir = acc_ref[2] / den
        l_mag = acc_ref[3] / den
        l_pid = acc_ref[4] / den
        l_chg = acc_ref[5] / den
        l_asn = acc_ref[0] / jnp.maximum(acc_ref[1], 1.0)
        l_stp = acc_ref[7] / float(_T * _B)
        out_ref[0] = (l_dir + l_mag + l_pid + 0.5 * l_chg + l_asn
                      + 0.5 * l_stp)
        out_ref[1] = l_dir
        out_ref[2] = l_mag
        out_ref[3] = l_pid
        out_ref[4] = l_chg
        out_ref[5] = l_asn
        out_ref[6] = l_stp


def kernel(pfo_momentum, pfo_p_mod, pfo_pid, pfo_charge, assignments,
           assignments_logits, stop_logits, gt_momentum, gt_p_mod, gt_pid,
           gt_charge, gt_batch, hit_to_pfo, hit_batch):
    del assignments
    lg = jnp.swapaxes(assignments_logits, 1, 2)
    hbc = hit_batch.astype(jnp.int32).reshape(1, _NH)
    htp2 = hit_to_pfo.astype(jnp.int32).reshape(1, _NH)
    gtb = gt_batch.astype(jnp.int32)
    gtbr = gtb.reshape(1, _NPFO)
    gtbc = gtb.reshape(_NPFO, 1)
    pmom = pfo_momentum.reshape(_T * _B, 3)
    ppm = pfo_p_mod.reshape(_T * _B, 1)
    ppid = pfo_pid.reshape(_T * _B, 5)
    pch = pfo_charge.reshape(_T * _B, 1)
    stp = stop_logits.reshape(_T * _B, 1)

    full = lambda s: pl.BlockSpec(s, lambda i: (0, 0))
    out = pl.pallas_call(
        _body,
        grid=(_NBLK,),
        in_specs=[
            pl.BlockSpec(memory_space=pltpu.MemorySpace.HBM),
            pl.BlockSpec((1, _C), lambda i: (0, i)),
            pl.BlockSpec((1, _C), lambda i: (0, i)),
            full((1, _NPFO)),
            full((_NPFO, 1)),
            full((_NPFO, 3)),
            full((_NPFO, 1)),
            full((_NPFO, 5)),
            full((_NPFO, 1)),
            full((_T * _B, 3)),
            full((_T * _B, 1)),
            full((_T * _B, 5)),
            full((_T * _B, 1)),
            full((_T * _B, 1)),
        ],
        out_specs=pl.BlockSpec(memory_space=pltpu.SMEM),
        out_shape=jax.ShapeDtypeStruct((8,), jnp.float32),
        scratch_shapes=[pltpu.SMEM((8,), jnp.float32),
                        pltpu.VMEM((8, _B), jnp.float32),
                        pltpu.VMEM((2, _T, _C), jnp.float32),
                        pltpu.VMEM((_T, _TAIL), jnp.float32),
                        pltpu.SemaphoreType.DMA((2,))],
        compiler_params=pltpu.CompilerParams(
            dimension_semantics=("arbitrary",)),
    )(lg, hbc, htp2, gtbr, gtbc, gt_momentum, gt_p_mod, gt_pid, gt_charge,
      pmom, ppm, ppid, pch, stp)
    return (out[0], out[1], out[2], out[3], out[4], out[5], out[6])
